# baseline (speedup 1.0000x reference)
import functools

import jax
import jax.numpy as jnp
from jax import lax
from jax.experimental import pallas as pl
from jax.experimental.pallas import tpu as pltpu
from jax.experimental.pallas import tpu_sc as plsc

_DIM = 300
_DIMP = 304
_NC = 2
_NS = 16
_NW = _NC * _NS
_CHUNK = 64
_OUTC = _CHUNK * _DIM
_NREG = _DIMP // 16
_TAIL = _DIM - (_NREG - 1) * 16


@functools.lru_cache(maxsize=None)
def _make_gather(B):
    assert B % (_NW * _CHUNK) == 0
    b_per_w = B // _NW
    nchunks = b_per_w // _CHUNK
    assert nchunks % 2 == 0
    mesh = plsc.VectorSubcoreMesh(core_axis_name="c", subcore_axis_name="s")

    @functools.partial(
        pl.kernel,
        mesh=mesh,
        out_type=jax.ShapeDtypeStruct((B * _DIM,), jnp.float32),
        scratch_types=[
            pltpu.VMEM((b_per_w,), jnp.int32),
            pltpu.VMEM((_CHUNK, _DIMP), jnp.float32),
            pltpu.VMEM((_CHUNK, _DIMP), jnp.float32),
            pltpu.VMEM((_OUTC,), jnp.float32),
            pltpu.VMEM((_OUTC,), jnp.float32),
            pltpu.SemaphoreType.DMA,
            pltpu.SemaphoreType.DMA,
            pltpu.SemaphoreType.DMA,
            pltpu.SemaphoreType.DMA,
        ],
        compiler_params=pltpu.CompilerParams(
            use_tc_tiling_on_sc=False, needs_layout_passes=False),
    )
    def gather(idx_hbm, table_hbm, out_hbm, idx_v, rows0, rows1,
               comp0, comp1, gsem0, gsem1, osem0, osem1):
        wid = lax.axis_index("s") * _NC + lax.axis_index("c")
        wbase = wid * b_per_w
        obase = wbase * _DIM
        pltpu.sync_copy(idx_hbm.at[pl.ds(wbase, b_per_w)], idx_v)
        rows = (rows0, rows1)
        comp = (comp0, comp1)
        gsems = (gsem0, gsem1)
        osems = (osem0, osem1)

        lane = lax.iota(jnp.int32, 16)
        tail_mask = lane < _TAIL

        def start_gather(j, b):
            pltpu.async_copy(
                table_hbm.at[idx_v.at[pl.ds(j * _CHUNK, _CHUNK)]],
                rows[b], gsems[b])

        def wait_gather(b):
            pltpu.make_async_copy(
                table_hbm.at[idx_v.at[pl.ds(0, _CHUNK)]], rows[b], gsems[b]
            ).wait()

        def start_out(j, b):
            pltpu.async_copy(
                comp[b], out_hbm.at[pl.ds(obase + j * _OUTC, _OUTC)], osems[b])

        def wait_out(b):
            pltpu.make_async_copy(
                comp[b], out_hbm.at[pl.ds(0, _OUTC)], osems[b]).wait()

        def compact(b):
            def row_body(r, didx):
                for k in range(_NREG - 1):
                    v = rows[b][r, pl.ds(16 * k, 16)]
                    plsc.store_scatter(comp[b], [didx], v)
                    didx = didx + 16
                v = rows[b][r, pl.ds((_NREG - 1) * 16, 16)]
                plsc.store_scatter(comp[b], [didx], v, mask=tail_mask)
                return didx + _TAIL
            lax.fori_loop(0, _CHUNK, row_body, lane)

        start_gather(0, 0)
        start_gather(1, 1)

        def body(jp, carry):
            for b in range(2):
                j = jp * 2 + b
                wait_gather(b)

                @pl.when(j >= 2)
                def _():
                    wait_out(b)

                compact(b)

                @pl.when(j + 2 < nchunks)
                def _():
                    start_gather(j + 2, b)

                start_out(j, b)
            return carry

        lax.fori_loop(0, nchunks // 2, body, 0)
        wait_out(0)
        wait_out(1)

    return gather


def kernel(x, table):
    B = x.shape[0] * x.shape[1]
    xi = x.reshape(B).astype(jnp.int32)
    tpad = jnp.pad(table, ((0, 0), (0, _DIMP - _DIM)))
    out = _make_gather(B)(xi, tpad)
    return out.reshape(x.shape[0], x.shape[1], _DIM)
